# baseline (speedup 1.0000x reference)
import functools

import jax
import jax.numpy as jnp
from jax import lax
from jax.experimental import pallas as pl
from jax.experimental.pallas import tpu as pltpu
from jax.experimental.pallas import tpu_sc as plsc

N = 10000
E = 160000
D_IN = 256
D_HID = 512
D_OUT = 64

NC = 2
NS = 16
CH = 125
CHUNK_ROWS = E // CH
SUB_CHUNKS = CHUNK_ROWS // NS
W_CHUNKS = CHUNK_ROWS // (NS * NC)
RB = 80
N_RB = N // RB
RB_ITERS = (N_RB + NS - 1) // NS

_MESH = plsc.VectorSubcoreMesh(core_axis_name="c", subcore_axis_name="s")


def _sc_aggregate(g_tbl, row2d, col2d, feat, edge_split):
    sub_chunks = CHUNK_ROWS // (NC * NS) if edge_split else SUB_CHUNKS
    rp = SUB_CHUNKS // 2
    passes = sub_chunks // rp

    @functools.partial(
        pl.kernel,
        out_type=jax.ShapeDtypeStruct((NC, N, feat), jnp.float32),
        mesh=_MESH,
        compiler_params=pltpu.CompilerParams(
            use_tc_tiling_on_sc=(feat % 128 == 0)),
        scratch_types=[
            pltpu.VMEM_SHARED((N, feat), jnp.float32),
            pltpu.VMEM((rp, CH), jnp.int32),
            pltpu.VMEM((rp, CH), jnp.int32),
            pltpu.VMEM((CH, feat), jnp.float32),
            pltpu.VMEM((CH, feat), jnp.float32),
            pltpu.SemaphoreType.DMA,
            pltpu.SemaphoreType.DMA,
            pltpu.SemaphoreType.DMA,
            pltpu.SemaphoreType.DMA,
        ],
    )
    def agg_kernel(g_hbm, row_hbm, col_hbm, out_hbm,
                   shared, rowbuf, colbuf, buf0, buf1,
                   sem0, sem1, ssem0, ssem1):
        c = lax.axis_index("c")
        s = lax.axis_index("s")
        tc = 0 if edge_split else c

        @pl.loop(0, RB_ITERS)
        def _(k):
            g = k * NS + s

            @pl.when(g < N_RB)
            def _():
                pltpu.sync_copy(g_hbm.at[tc, pl.ds(g * RB, RB)],
                                shared.at[pl.ds(g * RB, RB)])

        plsc.subcore_barrier()

        def gather(j, b, sm):
            pltpu.async_copy(g_hbm.at[tc].at[rowbuf.at[j]], b, sm)

        def wait_g(j, b, sm):
            pltpu.make_async_copy(g_hbm.at[tc].at[rowbuf.at[j]], b, sm).wait()

        def scatter(j, b, sm):
            pltpu.async_copy(b, shared.at[colbuf.at[j]], sm, add=True)

        def wait_s(j, b, sm):
            pltpu.make_async_copy(b, shared.at[colbuf.at[j]], sm).wait()

        sub_base = (c * NS + s) * sub_chunks if edge_split else s * sub_chunks

        @pl.loop(0, passes)
        def _(p):
            base = sub_base + p * rp
            pltpu.sync_copy(row_hbm.at[pl.ds(base, rp)], rowbuf)
            pltpu.sync_copy(col_hbm.at[pl.ds(base, rp)], colbuf)
            gather(0, buf0, sem0)
            gather(1, buf1, sem1)

            @pl.loop(0, rp, step=2)
            def _(j):
                wait_g(j, buf0, sem0)
                scatter(j, buf0, ssem0)
                wait_g(j + 1, buf1, sem1)
                scatter(j + 1, buf1, ssem1)

                @pl.when(j + 2 < rp)
                def _():
                    wait_s(j, buf0, ssem0)
                    gather(j + 2, buf0, sem0)
                    wait_s(j + 1, buf1, ssem1)
                    gather(j + 3, buf1, sem1)

            wait_s(rp - 2, buf0, ssem0)
            wait_s(rp - 1, buf1, ssem1)

        plsc.subcore_barrier()

        @pl.loop(0, RB_ITERS)
        def _(k):
            g = k * NS + s

            @pl.when(g < N_RB)
            def _():
                pltpu.sync_copy(shared.at[pl.ds(g * RB, RB)],
                                out_hbm.at[c, pl.ds(g * RB, RB)])

    return agg_kernel(g_tbl, row2d, col2d)


def _sc_degree(ones_small, col2d):
    @functools.partial(
        pl.kernel,
        out_type=jax.ShapeDtypeStruct((NC, N, 16), jnp.float32),
        mesh=_MESH,
        compiler_params=pltpu.CompilerParams(use_tc_tiling_on_sc=False),
        scratch_types=[
            pltpu.VMEM_SHARED((N, 16), jnp.float32),
            pltpu.VMEM((W_CHUNKS, CH), jnp.int32),
            pltpu.VMEM((CH, 16), jnp.float32),
        ],
    )
    def deg_kernel(ones_hbm, col_hbm, out_hbm, shared, colbuf, onesbuf):
        c = lax.axis_index("c")
        s = lax.axis_index("s")

        pltpu.sync_copy(ones_hbm, onesbuf)

        @pl.loop(0, RB_ITERS)
        def _(k):
            g = k * NS + s

            @pl.when(g < N_RB)
            def _():
                pltpu.sync_copy(ones_hbm.at[pl.ds(0, RB)],
                                shared.at[pl.ds(g * RB, RB)])

        w = s * NC + c
        pltpu.sync_copy(col_hbm.at[pl.ds(w * W_CHUNKS, W_CHUNKS)], colbuf)
        plsc.subcore_barrier()

        @pl.loop(0, W_CHUNKS, unroll=4)
        def _(j):
            pltpu.sync_copy(onesbuf, shared.at[colbuf.at[j]], add=True)

        plsc.subcore_barrier()

        @pl.loop(0, RB_ITERS)
        def _(k):
            g = k * NS + s

            @pl.when(g < N_RB)
            def _():
                pltpu.sync_copy(shared.at[pl.ds(g * RB, RB)],
                                out_hbm.at[c, pl.ds(g * RB, RB)])

    return deg_kernel(ones_small, col2d)


_BR = 1000


def _tc_prep(x, cnt):
    def body(x_ref, cnt_ref, g1_ref, dinv_ref):
        deg = cnt_ref[0] + cnt_ref[1] - 1.0
        dinv = lax.rsqrt(deg)
        dinv_ref[...] = dinv
        g = jax.nn.relu(x_ref[...]) * dinv[:, :1]
        g1_ref[0] = g[:, : D_IN // 2]
        g1_ref[1] = g[:, D_IN // 2:]

    return pl.pallas_call(
        body,
        grid=(N // _BR,),
        in_specs=[
            pl.BlockSpec((_BR, D_IN), lambda i: (i, 0)),
            pl.BlockSpec((NC, _BR, 16), lambda i: (0, i, 0)),
        ],
        out_specs=[
            pl.BlockSpec((NC, _BR, D_IN // 2), lambda i: (0, i, 0)),
            pl.BlockSpec((_BR, 16), lambda i: (i, 0)),
        ],
        out_shape=[
            jax.ShapeDtypeStruct((NC, N, D_IN // 2), jnp.float32),
            jax.ShapeDtypeStruct((N, 16), jnp.float32),
        ],
    )(x, cnt)


def _tc_mid(acc1, dinv16, W1, b1, W2):
    def body(acc_ref, dinv_ref, w1_ref, b1_ref, w2_ref, g2_ref):
        dinv = dinv_ref[:, :1]
        z = jnp.concatenate([acc_ref[0], acc_ref[1]], axis=1) * dinv
        h = jax.nn.relu(
            jnp.dot(z, w1_ref[...], preferred_element_type=jnp.float32)
            + b1_ref[...]
        )
        t = jnp.dot(h, w2_ref[...], preferred_element_type=jnp.float32)
        g2_ref[0] = t * dinv

    return pl.pallas_call(
        body,
        grid=(N // _BR,),
        in_specs=[
            pl.BlockSpec((NC, _BR, D_IN // 2), lambda i: (0, i, 0)),
            pl.BlockSpec((_BR, 16), lambda i: (i, 0)),
            pl.BlockSpec((D_IN, D_HID), lambda i: (0, 0)),
            pl.BlockSpec((1, D_HID), lambda i: (0, 0)),
            pl.BlockSpec((D_HID, D_OUT), lambda i: (0, 0)),
        ],
        out_specs=pl.BlockSpec((1, _BR, D_OUT), lambda i: (0, i, 0)),
        out_shape=jax.ShapeDtypeStruct((1, N, D_OUT), jnp.float32),
    )(acc1, dinv16, W1, b1, W2)


def _tc_final(acc2, g2, dinv16, b2):
    def body(acc_ref, g2_ref, dinv_ref, b2_ref, out_ref):
        dinv = dinv_ref[:, :1]
        out_ref[...] = (
            (acc_ref[0] + acc_ref[1] - g2_ref[0]) * dinv + b2_ref[...]
        )

    return pl.pallas_call(
        body,
        grid=(N // _BR,),
        in_specs=[
            pl.BlockSpec((NC, _BR, D_OUT), lambda i: (0, i, 0)),
            pl.BlockSpec((1, _BR, D_OUT), lambda i: (0, i, 0)),
            pl.BlockSpec((_BR, 16), lambda i: (i, 0)),
            pl.BlockSpec((1, D_OUT), lambda i: (0, 0)),
        ],
        out_specs=pl.BlockSpec((_BR, D_OUT), lambda i: (i, 0)),
        out_shape=jax.ShapeDtypeStruct((N, D_OUT), jnp.float32),
    )(acc2, g2, dinv16, b2)


def kernel(x, edge_index, W1, b1, W2, b2):
    edge_index = edge_index.astype(jnp.int32)
    row2d = edge_index[0].reshape(CHUNK_ROWS, CH)
    col2d = edge_index[1].reshape(CHUNK_ROWS, CH)

    ones_small = jnp.ones((CH, 16), jnp.float32)
    deg = _sc_degree(ones_small, col2d)
    g1, dinv16 = _tc_prep(x, deg)
    acc1 = _sc_aggregate(g1, row2d, col2d, D_IN // 2, edge_split=False)
    g2 = _tc_mid(acc1, dinv16, W1, b1.reshape(1, D_HID), W2)
    acc2 = _sc_aggregate(g2, row2d, col2d, D_OUT, edge_split=True)
    return _tc_final(acc2, g2, dinv16, b2.reshape(1, D_OUT))


# speedup vs baseline: 1.1664x; 1.1664x over previous
import functools

import jax
import jax.numpy as jnp
from jax import lax
from jax.experimental import pallas as pl
from jax.experimental.pallas import tpu as pltpu
from jax.experimental.pallas import tpu_sc as plsc

N = 10000
E = 160000
D_IN = 256
D_HID = 512
D_OUT = 64

NC = 2
NS = 16
CH = 125
CHUNK_ROWS = E // CH
SUB_CHUNKS = CHUNK_ROWS // NS
W_CHUNKS = CHUNK_ROWS // (NS * NC)
RB = 80
N_RB = N // RB
RB_ITERS = (N_RB + NS - 1) // NS

_MESH = plsc.VectorSubcoreMesh(core_axis_name="c", subcore_axis_name="s")


def _sc_aggregate(g_tbl, row2d, col2d, feat, edge_split):
    sub_chunks = CHUNK_ROWS // (NC * NS) if edge_split else SUB_CHUNKS
    rp = SUB_CHUNKS // 2
    passes = sub_chunks // rp

    @functools.partial(
        pl.kernel,
        out_type=jax.ShapeDtypeStruct((NC, N, feat), jnp.float32),
        mesh=_MESH,
        compiler_params=pltpu.CompilerParams(
            use_tc_tiling_on_sc=(feat % 128 == 0)),
        scratch_types=[
            pltpu.VMEM_SHARED((N, feat), jnp.float32),
            pltpu.VMEM((rp, CH), jnp.int32),
            pltpu.VMEM((rp, CH), jnp.int32),
            pltpu.VMEM((CH, feat), jnp.float32),
            pltpu.VMEM((CH, feat), jnp.float32),
            pltpu.SemaphoreType.DMA,
            pltpu.SemaphoreType.DMA,
        ],
    )
    def agg_kernel(g_hbm, row_hbm, col_hbm, out_hbm,
                   shared, rowbuf, colbuf, buf0, buf1, sem0, sem1):
        c = lax.axis_index("c")
        s = lax.axis_index("s")
        tc = 0 if edge_split else c

        @pl.loop(0, RB_ITERS)
        def _(k):
            g = k * NS + s

            @pl.when(g < N_RB)
            def _():
                pltpu.sync_copy(g_hbm.at[tc, pl.ds(g * RB, RB)],
                                shared.at[pl.ds(g * RB, RB)])

        plsc.subcore_barrier()

        def gather(j, b, sm):
            pltpu.async_copy(g_hbm.at[tc].at[rowbuf.at[j]], b, sm)

        def wait_g(j, b, sm):
            pltpu.make_async_copy(g_hbm.at[tc].at[rowbuf.at[j]], b, sm).wait()

        def scatter(j, b):
            pltpu.sync_copy(b, shared.at[colbuf.at[j]], add=True)

        sub_base = (c * NS + s) * sub_chunks if edge_split else s * sub_chunks

        @pl.loop(0, passes)
        def _(p):
            base = sub_base + p * rp
            pltpu.sync_copy(row_hbm.at[pl.ds(base, rp)], rowbuf)
            pltpu.sync_copy(col_hbm.at[pl.ds(base, rp)], colbuf)
            gather(0, buf0, sem0)

            @pl.loop(0, rp, step=2, unroll=2)
            def _(j):
                gather(j + 1, buf1, sem1)
                wait_g(j, buf0, sem0)
                scatter(j, buf0)

                @pl.when(j + 2 < rp)
                def _():
                    gather(j + 2, buf0, sem0)

                wait_g(j + 1, buf1, sem1)
                scatter(j + 1, buf1)

        plsc.subcore_barrier()

        @pl.loop(0, RB_ITERS)
        def _(k):
            g = k * NS + s

            @pl.when(g < N_RB)
            def _():
                pltpu.sync_copy(shared.at[pl.ds(g * RB, RB)],
                                out_hbm.at[c, pl.ds(g * RB, RB)])

    return agg_kernel(g_tbl, row2d, col2d)


def _sc_degree(ones_small, col2d):
    @functools.partial(
        pl.kernel,
        out_type=jax.ShapeDtypeStruct((NC, N, 16), jnp.float32),
        mesh=_MESH,
        compiler_params=pltpu.CompilerParams(use_tc_tiling_on_sc=False),
        scratch_types=[
            pltpu.VMEM_SHARED((N, 16), jnp.float32),
            pltpu.VMEM((W_CHUNKS, CH), jnp.int32),
            pltpu.VMEM((CH, 16), jnp.float32),
            pltpu.SemaphoreType.DMA,
        ],
    )
    def deg_kernel(ones_hbm, col_hbm, out_hbm, shared, colbuf, onesbuf, sem):
        c = lax.axis_index("c")
        s = lax.axis_index("s")

        pltpu.sync_copy(ones_hbm, onesbuf)

        @pl.loop(0, RB_ITERS)
        def _(k):
            g = k * NS + s

            @pl.when(g < N_RB)
            def _():
                pltpu.sync_copy(ones_hbm.at[pl.ds(0, RB)],
                                shared.at[pl.ds(g * RB, RB)])

        w = s * NC + c
        pltpu.sync_copy(col_hbm.at[pl.ds(w * W_CHUNKS, W_CHUNKS)], colbuf)
        plsc.subcore_barrier()

        @pl.loop(0, W_CHUNKS, step=4)
        def _(j):
            for k in range(4):
                pltpu.async_copy(onesbuf, shared.at[colbuf.at[j + k]], sem,
                                 add=True)
            for k in range(4):
                pltpu.make_async_copy(onesbuf, shared.at[colbuf.at[j + k]],
                                      sem).wait()

        plsc.subcore_barrier()

        @pl.loop(0, RB_ITERS)
        def _(k):
            g = k * NS + s

            @pl.when(g < N_RB)
            def _():
                pltpu.sync_copy(shared.at[pl.ds(g * RB, RB)],
                                out_hbm.at[c, pl.ds(g * RB, RB)])

    return deg_kernel(ones_small, col2d)


_BR = 1000


def _tc_prep(x, cnt):
    def body(x_ref, cnt_ref, g1_ref, dinv_ref):
        deg = cnt_ref[0] + cnt_ref[1] - 1.0
        dinv = lax.rsqrt(deg)
        dinv_ref[...] = dinv
        g = jax.nn.relu(x_ref[...]) * dinv[:, :1]
        g1_ref[0] = g[:, : D_IN // 2]
        g1_ref[1] = g[:, D_IN // 2:]

    return pl.pallas_call(
        body,
        grid=(N // _BR,),
        in_specs=[
            pl.BlockSpec((_BR, D_IN), lambda i: (i, 0)),
            pl.BlockSpec((NC, _BR, 16), lambda i: (0, i, 0)),
        ],
        out_specs=[
            pl.BlockSpec((NC, _BR, D_IN // 2), lambda i: (0, i, 0)),
            pl.BlockSpec((_BR, 16), lambda i: (i, 0)),
        ],
        out_shape=[
            jax.ShapeDtypeStruct((NC, N, D_IN // 2), jnp.float32),
            jax.ShapeDtypeStruct((N, 16), jnp.float32),
        ],
    )(x, cnt)


def _tc_mid(acc1, dinv16, W1, b1, W2):
    def body(acc_ref, dinv_ref, w1_ref, b1_ref, w2_ref, g2_ref):
        dinv = dinv_ref[:, :1]
        z = jnp.concatenate([acc_ref[0], acc_ref[1]], axis=1) * dinv
        h = jax.nn.relu(
            jnp.dot(z.astype(jnp.bfloat16), w1_ref[...],
                    preferred_element_type=jnp.float32)
            + b1_ref[...]
        )
        t = jnp.dot(h.astype(jnp.bfloat16), w2_ref[...],
                    preferred_element_type=jnp.float32)
        g2_ref[0] = t * dinv

    return pl.pallas_call(
        body,
        grid=(N // _BR,),
        in_specs=[
            pl.BlockSpec((NC, _BR, D_IN // 2), lambda i: (0, i, 0)),
            pl.BlockSpec((_BR, 16), lambda i: (i, 0)),
            pl.BlockSpec((D_IN, D_HID), lambda i: (0, 0)),
            pl.BlockSpec((1, D_HID), lambda i: (0, 0)),
            pl.BlockSpec((D_HID, D_OUT), lambda i: (0, 0)),
        ],
        out_specs=pl.BlockSpec((1, _BR, D_OUT), lambda i: (0, i, 0)),
        out_shape=jax.ShapeDtypeStruct((1, N, D_OUT), jnp.float32),
    )(acc1, dinv16, W1, b1, W2)


def _tc_final(acc2, g2, dinv16, b2):
    def body(acc_ref, g2_ref, dinv_ref, b2_ref, out_ref):
        dinv = dinv_ref[:, :1]
        out_ref[...] = (
            (acc_ref[0] + acc_ref[1] - g2_ref[0]) * dinv + b2_ref[...]
        )

    return pl.pallas_call(
        body,
        grid=(N // _BR,),
        in_specs=[
            pl.BlockSpec((NC, _BR, D_OUT), lambda i: (0, i, 0)),
            pl.BlockSpec((1, _BR, D_OUT), lambda i: (0, i, 0)),
            pl.BlockSpec((_BR, 16), lambda i: (i, 0)),
            pl.BlockSpec((1, D_OUT), lambda i: (0, 0)),
        ],
        out_specs=pl.BlockSpec((_BR, D_OUT), lambda i: (i, 0)),
        out_shape=jax.ShapeDtypeStruct((N, D_OUT), jnp.float32),
    )(acc2, g2, dinv16, b2)


def kernel(x, edge_index, W1, b1, W2, b2):
    edge_index = edge_index.astype(jnp.int32)
    row2d = edge_index[0].reshape(CHUNK_ROWS, CH)
    col2d = edge_index[1].reshape(CHUNK_ROWS, CH)

    ones_small = jnp.ones((CH, 16), jnp.float32)
    deg = _sc_degree(ones_small, col2d)
    g1, dinv16 = _tc_prep(x, deg)
    acc1 = _sc_aggregate(g1, row2d, col2d, D_IN // 2, edge_split=False)
    g2 = _tc_mid(acc1, dinv16, W1.astype(jnp.bfloat16),
                 b1.reshape(1, D_HID), W2.astype(jnp.bfloat16))
    acc2 = _sc_aggregate(g2, row2d, col2d, D_OUT, edge_split=True)
    return _tc_final(acc2, g2, dinv16, b2.reshape(1, D_OUT))


# speedup vs baseline: 1.2576x; 1.0782x over previous
import functools

import jax
import jax.numpy as jnp
from jax import lax
from jax.experimental import pallas as pl
from jax.experimental.pallas import tpu as pltpu
from jax.experimental.pallas import tpu_sc as plsc

N = 10000
E = 160000
D_IN = 256
D_HID = 512
D_OUT = 64

NC = 2
NS = 16
CH = 125
CHUNK_ROWS = E // CH
SUB_CHUNKS = CHUNK_ROWS // NS
W_CHUNKS = CHUNK_ROWS // (NS * NC)
RB = 80
N_RB = N // RB
RB_ITERS = (N_RB + NS - 1) // NS

_MESH = plsc.VectorSubcoreMesh(core_axis_name="c", subcore_axis_name="s")

_SLAB_A = 624
_SLAB_B = 632
_SLAB_SPLIT = 14


def _per_subcore_slab(s, copy_fn):
    @pl.when(s < _SLAB_SPLIT)
    def _():
        copy_fn(s * _SLAB_A, _SLAB_A)

    @pl.when(s >= _SLAB_SPLIT)
    def _():
        copy_fn(_SLAB_SPLIT * _SLAB_A + (s - _SLAB_SPLIT) * _SLAB_B, _SLAB_B)


def _sc_aggregate(g_tbl, row2d, col2d, feat, edge_split):
    sub_chunks = CHUNK_ROWS // (NC * NS) if edge_split else SUB_CHUNKS
    rp = SUB_CHUNKS // 2
    passes = sub_chunks // rp

    @functools.partial(
        pl.kernel,
        out_type=jax.ShapeDtypeStruct((NC, N, feat), jnp.float32),
        mesh=_MESH,
        compiler_params=pltpu.CompilerParams(
            use_tc_tiling_on_sc=(feat % 128 == 0)),
        scratch_types=[
            pltpu.VMEM_SHARED((N, feat), jnp.float32),
            pltpu.VMEM((rp, CH), jnp.int32),
            pltpu.VMEM((rp, CH), jnp.int32),
            pltpu.VMEM((CH, feat), jnp.float32),
            pltpu.VMEM((CH, feat), jnp.float32),
            pltpu.SemaphoreType.DMA,
            pltpu.SemaphoreType.DMA,
        ],
    )
    def agg_kernel(g_hbm, row_hbm, col_hbm, out_hbm,
                   shared, rowbuf, colbuf, buf0, buf1, sem0, sem1):
        c = lax.axis_index("c")
        s = lax.axis_index("s")
        tc = 0 if edge_split else c

        _per_subcore_slab(s, lambda off, rows: pltpu.sync_copy(
            g_hbm.at[tc, pl.ds(off, rows)], shared.at[pl.ds(off, rows)]))
        plsc.subcore_barrier()

        def gather(j, b, sm):
            pltpu.async_copy(g_hbm.at[tc].at[rowbuf.at[j]], b, sm)

        def wait_g(j, b, sm):
            pltpu.make_async_copy(g_hbm.at[tc].at[rowbuf.at[j]], b, sm).wait()

        def scatter(j, b):
            pltpu.sync_copy(b, shared.at[colbuf.at[j]], add=True)

        sub_base = (c * NS + s) * sub_chunks if edge_split else s * sub_chunks

        @pl.loop(0, passes)
        def _(p):
            base = sub_base + p * rp
            pltpu.sync_copy(row_hbm.at[pl.ds(base, rp)], rowbuf)
            pltpu.sync_copy(col_hbm.at[pl.ds(base, rp)], colbuf)
            gather(0, buf0, sem0)

            @pl.loop(0, rp, step=2, unroll=2)
            def _(j):
                gather(j + 1, buf1, sem1)
                wait_g(j, buf0, sem0)
                scatter(j, buf0)

                @pl.when(j + 2 < rp)
                def _():
                    gather(j + 2, buf0, sem0)

                wait_g(j + 1, buf1, sem1)
                scatter(j + 1, buf1)

        plsc.subcore_barrier()
        _per_subcore_slab(s, lambda off, rows: pltpu.sync_copy(
            shared.at[pl.ds(off, rows)], out_hbm.at[c, pl.ds(off, rows)]))

    return agg_kernel(g_tbl, row2d, col2d)


def _sc_degree(ones_small, col2d):
    @functools.partial(
        pl.kernel,
        out_type=jax.ShapeDtypeStruct((NC, N, 16), jnp.float32),
        mesh=_MESH,
        compiler_params=pltpu.CompilerParams(use_tc_tiling_on_sc=False),
        scratch_types=[
            pltpu.VMEM_SHARED((N, 16), jnp.float32),
            pltpu.VMEM((W_CHUNKS, CH), jnp.int32),
            pltpu.VMEM((CH, 16), jnp.float32),
            pltpu.SemaphoreType.DMA,
        ],
    )
    def deg_kernel(ones_hbm, col_hbm, out_hbm, shared, colbuf, onesbuf, sem):
        c = lax.axis_index("c")
        s = lax.axis_index("s")

        pltpu.sync_copy(ones_hbm.at[pl.ds(0, CH)], onesbuf)
        _per_subcore_slab(s, lambda off, rows: pltpu.sync_copy(
            ones_hbm.at[pl.ds(0, rows)], shared.at[pl.ds(off, rows)]))

        w = s * NC + c
        pltpu.sync_copy(col_hbm.at[pl.ds(w * W_CHUNKS, W_CHUNKS)], colbuf)
        plsc.subcore_barrier()

        @pl.loop(0, W_CHUNKS, step=4)
        def _(j):
            for k in range(4):
                pltpu.async_copy(onesbuf, shared.at[colbuf.at[j + k]], sem,
                                 add=True)
            for k in range(4):
                pltpu.make_async_copy(onesbuf, shared.at[colbuf.at[j + k]],
                                      sem).wait()

        plsc.subcore_barrier()
        _per_subcore_slab(s, lambda off, rows: pltpu.sync_copy(
            shared.at[pl.ds(off, rows)], out_hbm.at[c, pl.ds(off, rows)]))

    return deg_kernel(ones_small, col2d)


_BR = 1000


def _tc_prep(x, cnt):
    def body(x_ref, cnt_ref, g1_ref, dinv_ref):
        deg = cnt_ref[0] + cnt_ref[1] - 1.0
        dinv = lax.rsqrt(deg)
        dinv_ref[...] = dinv
        g = jax.nn.relu(x_ref[...]) * dinv[:, :1]
        g1_ref[0] = g[:, : D_IN // 2]
        g1_ref[1] = g[:, D_IN // 2:]

    return pl.pallas_call(
        body,
        grid=(N // _BR,),
        in_specs=[
            pl.BlockSpec((_BR, D_IN), lambda i: (i, 0)),
            pl.BlockSpec((NC, _BR, 16), lambda i: (0, i, 0)),
        ],
        out_specs=[
            pl.BlockSpec((NC, _BR, D_IN // 2), lambda i: (0, i, 0)),
            pl.BlockSpec((_BR, 16), lambda i: (i, 0)),
        ],
        out_shape=[
            jax.ShapeDtypeStruct((NC, N, D_IN // 2), jnp.float32),
            jax.ShapeDtypeStruct((N, 16), jnp.float32),
        ],
    )(x, cnt)


def _tc_mid(acc1, dinv16, W1, b1, W2):
    def body(acc_ref, dinv_ref, w1_ref, b1_ref, w2_ref, g2_ref):
        dinv = dinv_ref[:, :1]
        z = jnp.concatenate([acc_ref[0], acc_ref[1]], axis=1) * dinv
        h = jax.nn.relu(
            jnp.dot(z, w1_ref[...], preferred_element_type=jnp.float32)
            + b1_ref[...]
        )
        t = jnp.dot(h, w2_ref[...], preferred_element_type=jnp.float32)
        g2_ref[0] = t * dinv

    return pl.pallas_call(
        body,
        grid=(N // _BR,),
        in_specs=[
            pl.BlockSpec((NC, _BR, D_IN // 2), lambda i: (0, i, 0)),
            pl.BlockSpec((_BR, 16), lambda i: (i, 0)),
            pl.BlockSpec((D_IN, D_HID), lambda i: (0, 0)),
            pl.BlockSpec((1, D_HID), lambda i: (0, 0)),
            pl.BlockSpec((D_HID, D_OUT), lambda i: (0, 0)),
        ],
        out_specs=pl.BlockSpec((1, _BR, D_OUT), lambda i: (0, i, 0)),
        out_shape=jax.ShapeDtypeStruct((1, N, D_OUT), jnp.float32),
    )(acc1, dinv16, W1, b1, W2)


def _tc_final(acc2, g2, dinv16, b2):
    def body(acc_ref, g2_ref, dinv_ref, b2_ref, out_ref):
        dinv = dinv_ref[:, :1]
        out_ref[...] = (
            (acc_ref[0] + acc_ref[1] - g2_ref[0]) * dinv + b2_ref[...]
        )

    return pl.pallas_call(
        body,
        grid=(N // _BR,),
        in_specs=[
            pl.BlockSpec((NC, _BR, D_OUT), lambda i: (0, i, 0)),
            pl.BlockSpec((1, _BR, D_OUT), lambda i: (0, i, 0)),
            pl.BlockSpec((_BR, 16), lambda i: (i, 0)),
            pl.BlockSpec((1, D_OUT), lambda i: (0, 0)),
        ],
        out_specs=pl.BlockSpec((_BR, D_OUT), lambda i: (i, 0)),
        out_shape=jax.ShapeDtypeStruct((N, D_OUT), jnp.float32),
    )(acc2, g2, dinv16, b2)


def kernel(x, edge_index, W1, b1, W2, b2):
    edge_index = edge_index.astype(jnp.int32)
    row2d = edge_index[0].reshape(CHUNK_ROWS, CH)
    col2d = edge_index[1].reshape(CHUNK_ROWS, CH)

    ones_small = jnp.ones((_SLAB_B, 16), jnp.float32)
    deg = _sc_degree(ones_small, col2d)
    g1, dinv16 = _tc_prep(x, deg)
    acc1 = _sc_aggregate(g1, row2d, col2d, D_IN // 2, edge_split=False)
    g2 = _tc_mid(acc1, dinv16, W1, b1.reshape(1, D_HID), W2)
    acc2 = _sc_aggregate(g2, row2d, col2d, D_OUT, edge_split=True)
    return _tc_final(acc2, g2, dinv16, b2.reshape(1, D_OUT))


# speedup vs baseline: 1.2887x; 1.0247x over previous
import functools

import jax
import jax.numpy as jnp
from jax import lax
from jax.experimental import pallas as pl
from jax.experimental.pallas import tpu as pltpu
from jax.experimental.pallas import tpu_sc as plsc

N = 10000
E = 160000
D_IN = 256
D_HID = 512
D_OUT = 64

NC = 2
NS = 16
CH = 125
CHUNK_ROWS = E // CH
SUB_CHUNKS = CHUNK_ROWS // NS
W_CHUNKS = CHUNK_ROWS // (NS * NC)
RB = 80
N_RB = N // RB
RB_ITERS = (N_RB + NS - 1) // NS

_MESH = plsc.VectorSubcoreMesh(core_axis_name="c", subcore_axis_name="s")

_SLAB_A = 624
_SLAB_B = 632
_SLAB_SPLIT = 14


def _per_subcore_slab(s, copy_fn):
    @pl.when(s < _SLAB_SPLIT)
    def _():
        copy_fn(s * _SLAB_A, _SLAB_A)

    @pl.when(s >= _SLAB_SPLIT)
    def _():
        copy_fn(_SLAB_SPLIT * _SLAB_A + (s - _SLAB_SPLIT) * _SLAB_B, _SLAB_B)


def _sc_aggregate(g_tbl, row2d, col2d, feat, edge_split):
    sub_chunks = CHUNK_ROWS // (NC * NS) if edge_split else SUB_CHUNKS
    rp = SUB_CHUNKS // 2
    passes = sub_chunks // rp

    @functools.partial(
        pl.kernel,
        out_type=jax.ShapeDtypeStruct((NC, N, feat), jnp.float32),
        mesh=_MESH,
        compiler_params=pltpu.CompilerParams(
            use_tc_tiling_on_sc=(feat % 128 == 0)),
        scratch_types=[
            pltpu.VMEM_SHARED((N, feat), jnp.float32),
            pltpu.VMEM((rp, CH), jnp.int32),
            pltpu.VMEM((rp, CH), jnp.int32),
            pltpu.VMEM((CH, feat), jnp.float32),
            pltpu.VMEM((CH, feat), jnp.float32),
            pltpu.SemaphoreType.DMA,
            pltpu.SemaphoreType.DMA,
        ],
    )
    def agg_kernel(g_hbm, row_hbm, col_hbm, out_hbm,
                   shared, rowbuf, colbuf, buf0, buf1, sem0, sem1):
        c = lax.axis_index("c")
        s = lax.axis_index("s")
        tc = 0 if edge_split else c

        _per_subcore_slab(s, lambda off, rows: pltpu.sync_copy(
            g_hbm.at[tc, pl.ds(off, rows)], shared.at[pl.ds(off, rows)]))
        plsc.subcore_barrier()

        def gather(j, b, sm):
            pltpu.async_copy(g_hbm.at[tc].at[rowbuf.at[j]], b, sm)

        def wait_g(j, b, sm):
            pltpu.make_async_copy(g_hbm.at[tc].at[rowbuf.at[j]], b, sm).wait()

        def scatter(j, b):
            pltpu.sync_copy(b, shared.at[colbuf.at[j]], add=True)

        sub_base = (c * NS + s) * sub_chunks if edge_split else s * sub_chunks

        @pl.loop(0, passes)
        def _(p):
            base = sub_base + p * rp
            pltpu.sync_copy(row_hbm.at[pl.ds(base, rp)], rowbuf)
            pltpu.sync_copy(col_hbm.at[pl.ds(base, rp)], colbuf)
            gather(0, buf0, sem0)

            @pl.loop(0, rp, step=2, unroll=2)
            def _(j):
                gather(j + 1, buf1, sem1)
                wait_g(j, buf0, sem0)
                scatter(j, buf0)

                @pl.when(j + 2 < rp)
                def _():
                    gather(j + 2, buf0, sem0)

                wait_g(j + 1, buf1, sem1)
                scatter(j + 1, buf1)

        plsc.subcore_barrier()
        _per_subcore_slab(s, lambda off, rows: pltpu.sync_copy(
            shared.at[pl.ds(off, rows)], out_hbm.at[c, pl.ds(off, rows)]))

    return agg_kernel(g_tbl, row2d, col2d)


def _sc_degree(ones_small, col2d):
    @functools.partial(
        pl.kernel,
        out_type=jax.ShapeDtypeStruct((NC, N, 16), jnp.float32),
        mesh=_MESH,
        compiler_params=pltpu.CompilerParams(use_tc_tiling_on_sc=False),
        scratch_types=[
            pltpu.VMEM_SHARED((N, 16), jnp.float32),
            pltpu.VMEM((W_CHUNKS, CH), jnp.int32),
            pltpu.VMEM((CH, 16), jnp.float32),
            pltpu.SemaphoreType.DMA,
        ],
    )
    def deg_kernel(ones_hbm, col_hbm, out_hbm, shared, colbuf, onesbuf, sem):
        c = lax.axis_index("c")
        s = lax.axis_index("s")

        pltpu.sync_copy(ones_hbm.at[pl.ds(0, CH)], onesbuf)
        _per_subcore_slab(s, lambda off, rows: pltpu.sync_copy(
            ones_hbm.at[pl.ds(0, rows)], shared.at[pl.ds(off, rows)]))

        w = s * NC + c
        pltpu.sync_copy(col_hbm.at[pl.ds(w * W_CHUNKS, W_CHUNKS)], colbuf)
        plsc.subcore_barrier()

        @pl.loop(0, W_CHUNKS, step=4)
        def _(j):
            for k in range(4):
                pltpu.async_copy(onesbuf, shared.at[colbuf.at[j + k]], sem,
                                 add=True)
            for k in range(4):
                pltpu.make_async_copy(onesbuf, shared.at[colbuf.at[j + k]],
                                      sem).wait()

        plsc.subcore_barrier()
        _per_subcore_slab(s, lambda off, rows: pltpu.sync_copy(
            shared.at[pl.ds(off, rows)], out_hbm.at[c, pl.ds(off, rows)]))

    return deg_kernel(ones_small, col2d)


_BR = 2000


def _tc_prep(x, cnt):
    def body(x_ref, cnt_ref, g1_ref, dinv_ref):
        deg = cnt_ref[0] + cnt_ref[1] - 1.0
        dinv = lax.rsqrt(deg)
        dinv_ref[...] = dinv
        g = jax.nn.relu(x_ref[...]) * dinv[:, :1]
        g1_ref[0] = g[:, : D_IN // 2]
        g1_ref[1] = g[:, D_IN // 2:]

    return pl.pallas_call(
        body,
        grid=(N // _BR,),
        in_specs=[
            pl.BlockSpec((_BR, D_IN), lambda i: (i, 0)),
            pl.BlockSpec((NC, _BR, 16), lambda i: (0, i, 0)),
        ],
        out_specs=[
            pl.BlockSpec((NC, _BR, D_IN // 2), lambda i: (0, i, 0)),
            pl.BlockSpec((_BR, 16), lambda i: (i, 0)),
        ],
        out_shape=[
            jax.ShapeDtypeStruct((NC, N, D_IN // 2), jnp.float32),
            jax.ShapeDtypeStruct((N, 16), jnp.float32),
        ],
    )(x, cnt)


def _tc_mid(acc1, dinv16, W1, b1, W2):
    def body(acc_ref, dinv_ref, w1_ref, b1_ref, w2_ref, g2_ref):
        dinv = dinv_ref[:, :1]
        z = jnp.concatenate([acc_ref[0], acc_ref[1]], axis=1) * dinv
        h = jax.nn.relu(
            jnp.dot(z, w1_ref[...], preferred_element_type=jnp.float32)
            + b1_ref[...]
        )
        t = jnp.dot(h, w2_ref[...], preferred_element_type=jnp.float32)
        g2_ref[0] = t * dinv

    return pl.pallas_call(
        body,
        grid=(N // _BR,),
        in_specs=[
            pl.BlockSpec((NC, _BR, D_IN // 2), lambda i: (0, i, 0)),
            pl.BlockSpec((_BR, 16), lambda i: (i, 0)),
            pl.BlockSpec((D_IN, D_HID), lambda i: (0, 0)),
            pl.BlockSpec((1, D_HID), lambda i: (0, 0)),
            pl.BlockSpec((D_HID, D_OUT), lambda i: (0, 0)),
        ],
        out_specs=pl.BlockSpec((1, _BR, D_OUT), lambda i: (0, i, 0)),
        out_shape=jax.ShapeDtypeStruct((1, N, D_OUT), jnp.float32),
    )(acc1, dinv16, W1, b1, W2)


def _tc_final(acc2, g2, dinv16, b2):
    def body(acc_ref, g2_ref, dinv_ref, b2_ref, out_ref):
        dinv = dinv_ref[:, :1]
        out_ref[...] = (
            (acc_ref[0] + acc_ref[1] - g2_ref[0]) * dinv + b2_ref[...]
        )

    return pl.pallas_call(
        body,
        grid=(N // _BR,),
        in_specs=[
            pl.BlockSpec((NC, _BR, D_OUT), lambda i: (0, i, 0)),
            pl.BlockSpec((1, _BR, D_OUT), lambda i: (0, i, 0)),
            pl.BlockSpec((_BR, 16), lambda i: (i, 0)),
            pl.BlockSpec((1, D_OUT), lambda i: (0, 0)),
        ],
        out_specs=pl.BlockSpec((_BR, D_OUT), lambda i: (i, 0)),
        out_shape=jax.ShapeDtypeStruct((N, D_OUT), jnp.float32),
    )(acc2, g2, dinv16, b2)


def kernel(x, edge_index, W1, b1, W2, b2):
    edge_index = edge_index.astype(jnp.int32)
    row2d = edge_index[0].reshape(CHUNK_ROWS, CH)
    col2d = edge_index[1].reshape(CHUNK_ROWS, CH)

    ones_small = jnp.ones((_SLAB_B, 16), jnp.float32)
    deg = _sc_degree(ones_small, col2d)
    g1, dinv16 = _tc_prep(x, deg)
    acc1 = _sc_aggregate(g1, row2d, col2d, D_IN // 2, edge_split=False)
    g2 = _tc_mid(acc1, dinv16, W1, b1.reshape(1, D_HID), W2)
    acc2 = _sc_aggregate(g2, row2d, col2d, D_OUT, edge_split=True)
    return _tc_final(acc2, g2, dinv16, b2.reshape(1, D_OUT))


# speedup vs baseline: 1.3313x; 1.0331x over previous
import functools

import jax
import jax.numpy as jnp
from jax import lax
from jax.experimental import pallas as pl
from jax.experimental.pallas import tpu as pltpu
from jax.experimental.pallas import tpu_sc as plsc

N = 10000
E = 160000
D_IN = 256
D_HID = 512
D_OUT = 64

NC = 2
NS = 16
CH = 125
CHUNK_ROWS = E // CH
SUB_CHUNKS = CHUNK_ROWS // NS
W_CHUNKS = CHUNK_ROWS // (NS * NC)
RB = 80
N_RB = N // RB
RB_ITERS = (N_RB + NS - 1) // NS

_MESH = plsc.VectorSubcoreMesh(core_axis_name="c", subcore_axis_name="s")

_SLAB_A = 624
_SLAB_B = 632
_SLAB_SPLIT = 14


def _per_subcore_slab(s, copy_fn):
    @pl.when(s < _SLAB_SPLIT)
    def _():
        copy_fn(s * _SLAB_A, _SLAB_A)

    @pl.when(s >= _SLAB_SPLIT)
    def _():
        copy_fn(_SLAB_SPLIT * _SLAB_A + (s - _SLAB_SPLIT) * _SLAB_B, _SLAB_B)


def _sc_aggregate(g_tbl, row2d, col2d, feat, edge_split):
    sub_chunks = CHUNK_ROWS // (NC * NS) if edge_split else SUB_CHUNKS
    rp = SUB_CHUNKS // 2
    passes = sub_chunks // rp
    nbuf = 4 if edge_split else 2

    @functools.partial(
        pl.kernel,
        out_type=jax.ShapeDtypeStruct((NC, N, feat), jnp.float32),
        mesh=_MESH,
        compiler_params=pltpu.CompilerParams(
            use_tc_tiling_on_sc=(feat % 128 == 0)),
        scratch_types=[
            pltpu.VMEM_SHARED((N, feat), jnp.float32),
            pltpu.VMEM((rp, CH), jnp.int32),
            pltpu.VMEM((rp, CH), jnp.int32),
        ] + [pltpu.VMEM((CH, feat), jnp.float32)] * nbuf
          + [pltpu.SemaphoreType.DMA] * nbuf,
    )
    def agg_kernel(g_hbm, row_hbm, col_hbm, out_hbm,
                   shared, rowbuf, colbuf, *bufsem):
        bufs = bufsem[:nbuf]
        sems = bufsem[nbuf:]
        c = lax.axis_index("c")
        s = lax.axis_index("s")
        tc = 0 if edge_split else c

        _per_subcore_slab(s, lambda off, rows: pltpu.sync_copy(
            g_hbm.at[tc, pl.ds(off, rows)], shared.at[pl.ds(off, rows)]))
        plsc.subcore_barrier()

        def gather(j, k):
            pltpu.async_copy(g_hbm.at[tc].at[rowbuf.at[j]], bufs[k], sems[k])

        def wait_g(j, k):
            pltpu.make_async_copy(g_hbm.at[tc].at[rowbuf.at[j]], bufs[k],
                                  sems[k]).wait()

        def scatter(j, k):
            pltpu.sync_copy(bufs[k], shared.at[colbuf.at[j]], add=True)

        sub_base = (c * NS + s) * sub_chunks if edge_split else s * sub_chunks

        @pl.loop(0, passes)
        def _(p):
            base = sub_base + p * rp
            pltpu.sync_copy(row_hbm.at[pl.ds(base, rp)], rowbuf)
            pltpu.sync_copy(col_hbm.at[pl.ds(base, rp)], colbuf)
            for k in range(nbuf - 1):
                gather(k, k)

            @pl.loop(0, rp, step=nbuf)
            def _(j):
                for k in range(nbuf):
                    gather_j = j + k + nbuf - 1
                    if k == 0:
                        gather(gather_j, nbuf - 1)
                    else:
                        @pl.when(gather_j < rp)
                        def _(gj=gather_j, kk=k - 1):
                            gather(gj, kk)

                    wait_g(j + k, k)
                    scatter(j + k, k)

        plsc.subcore_barrier()
        _per_subcore_slab(s, lambda off, rows: pltpu.sync_copy(
            shared.at[pl.ds(off, rows)], out_hbm.at[c, pl.ds(off, rows)]))

    return agg_kernel(g_tbl, row2d, col2d)


def _sc_degree(ones_small, col2d):
    @functools.partial(
        pl.kernel,
        out_type=jax.ShapeDtypeStruct((NC, N, 16), jnp.float32),
        mesh=_MESH,
        compiler_params=pltpu.CompilerParams(use_tc_tiling_on_sc=False),
        scratch_types=[
            pltpu.VMEM_SHARED((N, 16), jnp.float32),
            pltpu.VMEM((W_CHUNKS, CH), jnp.int32),
            pltpu.VMEM((CH, 16), jnp.float32),
            pltpu.SemaphoreType.DMA,
        ],
    )
    def deg_kernel(ones_hbm, col_hbm, out_hbm, shared, colbuf, onesbuf, sem):
        c = lax.axis_index("c")
        s = lax.axis_index("s")

        pltpu.sync_copy(ones_hbm.at[pl.ds(0, CH)], onesbuf)
        _per_subcore_slab(s, lambda off, rows: pltpu.sync_copy(
            ones_hbm.at[pl.ds(0, rows)], shared.at[pl.ds(off, rows)]))

        w = s * NC + c
        pltpu.sync_copy(col_hbm.at[pl.ds(w * W_CHUNKS, W_CHUNKS)], colbuf)
        plsc.subcore_barrier()

        @pl.loop(0, W_CHUNKS, step=4)
        def _(j):
            for k in range(4):
                pltpu.async_copy(onesbuf, shared.at[colbuf.at[j + k]], sem,
                                 add=True)
            for k in range(4):
                pltpu.make_async_copy(onesbuf, shared.at[colbuf.at[j + k]],
                                      sem).wait()

        plsc.subcore_barrier()
        _per_subcore_slab(s, lambda off, rows: pltpu.sync_copy(
            shared.at[pl.ds(off, rows)], out_hbm.at[c, pl.ds(off, rows)]))

    return deg_kernel(ones_small, col2d)


_BR = 2000


def _tc_prep(x, cnt):
    def body(x_ref, cnt_ref, g1_ref, dinv_ref):
        deg = cnt_ref[0] + cnt_ref[1] - 1.0
        dinv = lax.rsqrt(deg)
        dinv_ref[...] = dinv
        g = jax.nn.relu(x_ref[...]) * dinv[:, :1]
        g1_ref[0] = g[:, : D_IN // 2]
        g1_ref[1] = g[:, D_IN // 2:]

    return pl.pallas_call(
        body,
        grid=(N // _BR,),
        in_specs=[
            pl.BlockSpec((_BR, D_IN), lambda i: (i, 0)),
            pl.BlockSpec((NC, _BR, 16), lambda i: (0, i, 0)),
        ],
        out_specs=[
            pl.BlockSpec((NC, _BR, D_IN // 2), lambda i: (0, i, 0)),
            pl.BlockSpec((_BR, 16), lambda i: (i, 0)),
        ],
        out_shape=[
            jax.ShapeDtypeStruct((NC, N, D_IN // 2), jnp.float32),
            jax.ShapeDtypeStruct((N, 16), jnp.float32),
        ],
    )(x, cnt)


def _tc_mid(acc1, dinv16, W1, b1, W2):
    def body(acc_ref, dinv_ref, w1_ref, b1_ref, w2_ref, g2_ref):
        dinv = dinv_ref[:, :1]
        z = jnp.concatenate([acc_ref[0], acc_ref[1]], axis=1) * dinv
        h = jax.nn.relu(
            jnp.dot(z, w1_ref[...], preferred_element_type=jnp.float32)
            + b1_ref[...]
        )
        t = jnp.dot(h, w2_ref[...], preferred_element_type=jnp.float32)
        g2_ref[0] = t * dinv

    return pl.pallas_call(
        body,
        grid=(N // _BR,),
        in_specs=[
            pl.BlockSpec((NC, _BR, D_IN // 2), lambda i: (0, i, 0)),
            pl.BlockSpec((_BR, 16), lambda i: (i, 0)),
            pl.BlockSpec((D_IN, D_HID), lambda i: (0, 0)),
            pl.BlockSpec((1, D_HID), lambda i: (0, 0)),
            pl.BlockSpec((D_HID, D_OUT), lambda i: (0, 0)),
        ],
        out_specs=pl.BlockSpec((1, _BR, D_OUT), lambda i: (0, i, 0)),
        out_shape=jax.ShapeDtypeStruct((1, N, D_OUT), jnp.float32),
    )(acc1, dinv16, W1, b1, W2)


def _tc_final(acc2, g2, dinv16, b2):
    def body(acc_ref, g2_ref, dinv_ref, b2_ref, out_ref):
        dinv = dinv_ref[:, :1]
        out_ref[...] = (
            (acc_ref[0] + acc_ref[1] - g2_ref[0]) * dinv + b2_ref[...]
        )

    return pl.pallas_call(
        body,
        grid=(N // _BR,),
        in_specs=[
            pl.BlockSpec((NC, _BR, D_OUT), lambda i: (0, i, 0)),
            pl.BlockSpec((1, _BR, D_OUT), lambda i: (0, i, 0)),
            pl.BlockSpec((_BR, 16), lambda i: (i, 0)),
            pl.BlockSpec((1, D_OUT), lambda i: (0, 0)),
        ],
        out_specs=pl.BlockSpec((_BR, D_OUT), lambda i: (i, 0)),
        out_shape=jax.ShapeDtypeStruct((N, D_OUT), jnp.float32),
    )(acc2, g2, dinv16, b2)


def kernel(x, edge_index, W1, b1, W2, b2):
    edge_index = edge_index.astype(jnp.int32)
    row2d = edge_index[0].reshape(CHUNK_ROWS, CH)
    col2d = edge_index[1].reshape(CHUNK_ROWS, CH)

    ones_small = jnp.ones((_SLAB_B, 16), jnp.float32)
    deg = _sc_degree(ones_small, col2d)
    g1, dinv16 = _tc_prep(x, deg)
    acc1 = _sc_aggregate(g1, row2d, col2d, D_IN // 2, edge_split=False)
    g2 = _tc_mid(acc1, dinv16, W1, b1.reshape(1, D_HID), W2)
    acc2 = _sc_aggregate(g2, row2d, col2d, D_OUT, edge_split=True)
    return _tc_final(acc2, g2, dinv16, b2.reshape(1, D_OUT))
